# Initial kernel scaffold; baseline (speedup 1.0000x reference)
import jax
import jax.numpy as jnp
from jax.experimental import pallas as pl


def kernel(pos, species, edge_index, params):
    raise NotImplementedError("write your pallas kernel here")


# speedup vs baseline: 4.7492x; 4.7492x over previous
import functools

import jax
import jax.numpy as jnp
import numpy as np
from jax.experimental import pallas as pl

_CUTOFF = 4.5
_C = 32
_NUM_DEGREES = 3
_OUTC = _C * _NUM_DEGREES
_HEADS = 8
_DATTN = 16
_HD = _DATTN // _HEADS
_INV_SQRT_HD = 1.0 / np.sqrt(_HD)

_DX = _CUTOFF / (_C - 2)


def _m16x8():
    row = jax.lax.broadcasted_iota(jnp.int32, (_DATTN, _HEADS), 0) // _HD
    col = jax.lax.broadcasted_iota(jnp.int32, (_DATTN, _HEADS), 1)
    return (row == col).astype(jnp.float32)


def _m8x16():
    row = jax.lax.broadcasted_iota(jnp.int32, (_HEADS, _DATTN), 0)
    col = jax.lax.broadcasted_iota(jnp.int32, (_HEADS, _DATTN), 1) // _HD
    return (row == col).astype(jnp.float32)


def _centers():
    return jax.lax.broadcasted_iota(
        jnp.int32, (1, _C - 1), 1).astype(jnp.float32) * _DX


def _edge_tile(e):
    for te in (4000, 2048, 1024, 512, 256, 128, 64, 32, 16, 8):
        if e % te == 0:
            return te
    return e


def _edge_grid(e):
    te = _edge_tile(e)
    return te, e // te


def _espec(te, width):
    return pl.BlockSpec((te, width), lambda i: (i, 0))


def _wspec(shape):
    nd = len(shape)
    return pl.BlockSpec(shape, lambda i: (0,) * nd)


def _geom_fwd_body(rel_ref, scale_ref, ef_ref):
    rel = rel_ref[...]
    d2 = jnp.sum(rel * rel, axis=1, keepdims=True) + 1e-12
    d = jnp.sqrt(d2)
    inside = d < _CUTOFF
    xc = jnp.where(inside, d / _CUTOFF, 0.5)
    bump = jnp.exp(1.0 - 1.0 / (1.0 - xc * xc))
    smooth = jnp.exp(-1.0 / jnp.maximum(d, 1e-9))
    scale_ref[...] = jnp.where(inside, bump * smooth, 0.0)
    rb = jnp.exp(-2.0 * (d - _centers()) ** 2 / _DX**2)
    r_soft = jnp.sqrt(d2 + 1.0) - 1.0
    ef_ref[...] = jnp.concatenate([rb, r_soft], axis=1)


def _geom_bwd_body(rel_ref, dscale_ref, def_ref, drel_ref):
    rel = rel_ref[...]
    d2 = jnp.sum(rel * rel, axis=1, keepdims=True) + 1e-12
    d = jnp.sqrt(d2)
    inside = d < _CUTOFF
    xc = jnp.where(inside, d / _CUTOFF, 0.5)
    bump = jnp.exp(1.0 - 1.0 / (1.0 - xc * xc))
    smooth = jnp.exp(-1.0 / jnp.maximum(d, 1e-9))
    one_m = 1.0 - xc * xc
    g_scale = jnp.where(
        inside,
        bump * smooth * (1.0 / d2 - 2.0 * xc / (_CUTOFF * one_m * one_m)),
        0.0,
    )
    rb = jnp.exp(-2.0 * (d - _centers()) ** 2 / _DX**2)
    g_rb = rb * (-4.0 * (d - _centers()) / _DX**2)
    g_rsoft = d / jnp.sqrt(d2 + 1.0)
    def_ = def_ref[...]
    dd = (
        dscale_ref[...] * g_scale
        + jnp.sum(def_[:, : _C - 1] * g_rb, axis=1, keepdims=True)
        + def_[:, _C - 1 : _C] * g_rsoft
    )
    drel_ref[...] = rel * (dd / d)


@jax.custom_vjp
def _geom(rel):
    te, nb = _edge_grid(rel.shape[0])
    return pl.pallas_call(
        _geom_fwd_body,
        grid=(nb,),
        in_specs=[_espec(te, 3)],
        out_specs=[_espec(te, 1), _espec(te, _C)],
        out_shape=[
            jax.ShapeDtypeStruct((rel.shape[0], 1), jnp.float32),
            jax.ShapeDtypeStruct((rel.shape[0], _C), jnp.float32),
        ],
    )(rel)


def _geom_vfwd(rel):
    return _geom(rel), (rel,)


def _geom_vbwd(res, ct):
    (rel,) = res
    dscale, def_ = ct
    te, nb = _edge_grid(rel.shape[0])
    drel = pl.pallas_call(
        _geom_bwd_body,
        grid=(nb,),
        in_specs=[_espec(te, 3), _espec(te, 1), _espec(te, _C)],
        out_specs=_espec(te, 3),
        out_shape=jax.ShapeDtypeStruct(rel.shape, jnp.float32),
    )(rel, dscale, def_)
    return (drel,)


_geom.defvjp(_geom_vfwd, _geom_vbwd)


def _attn_fwd_body(xs_ref, qd_ref, ef_ref, wk_ref, wv_ref, wek_ref, wev_ref,
                   logits_ref, v_ref):
    xs = xs_ref[...]
    ef = ef_ref[...]
    k = jnp.dot(xs, wk_ref[...], preferred_element_type=jnp.float32) + jnp.dot(
        ef, wek_ref[...], preferred_element_type=jnp.float32)
    v = jnp.dot(xs, wv_ref[...], preferred_element_type=jnp.float32) + jnp.dot(
        ef, wev_ref[...], preferred_element_type=jnp.float32)
    logits_ref[...] = jnp.dot(
        qd_ref[...] * k, _m16x8(), preferred_element_type=jnp.float32) * _INV_SQRT_HD
    v_ref[...] = v


def _attn_bwd_body(xs_ref, qd_ref, ef_ref, wk_ref, wek_ref,
                   wvt_ref, wevt_ref, wkt_ref, wekt_ref,
                   dlogits_ref, dv_ref,
                   dxs_ref, dqd_ref, def_ref):
    xs = xs_ref[...]
    ef = ef_ref[...]
    k = jnp.dot(xs, wk_ref[...], preferred_element_type=jnp.float32) + jnp.dot(
        ef, wek_ref[...], preferred_element_type=jnp.float32)
    dl16 = jnp.dot(dlogits_ref[...] * _INV_SQRT_HD, _m8x16(),
                   preferred_element_type=jnp.float32)
    dqd_ref[...] = dl16 * k
    dk = dl16 * qd_ref[...]
    dv = dv_ref[...]
    dxs_ref[...] = jnp.dot(dk, wkt_ref[...], preferred_element_type=jnp.float32) + jnp.dot(
        dv, wvt_ref[...], preferred_element_type=jnp.float32)
    def_ref[...] = jnp.dot(dk, wekt_ref[...], preferred_element_type=jnp.float32) + jnp.dot(
        dv, wevt_ref[...], preferred_element_type=jnp.float32)


@jax.custom_vjp
def _attn(xs, qd, ef, wk, wv, wek, wev):
    e = xs.shape[0]
    te, nb = _edge_grid(e)
    return pl.pallas_call(
        _attn_fwd_body,
        grid=(nb,),
        in_specs=[
            _espec(te, _C), _espec(te, _DATTN), _espec(te, _C),
            _wspec((_C, _DATTN)), _wspec((_C, _DATTN)),
            _wspec((_C, _DATTN)), _wspec((_C, _DATTN)),
        ],
        out_specs=[_espec(te, _HEADS), _espec(te, _DATTN)],
        out_shape=[
            jax.ShapeDtypeStruct((e, _HEADS), jnp.float32),
            jax.ShapeDtypeStruct((e, _DATTN), jnp.float32),
        ],
    )(xs, qd, ef, wk, wv, wek, wev)


def _attn_vfwd(xs, qd, ef, wk, wv, wek, wev):
    return _attn(xs, qd, ef, wk, wv, wek, wev), (xs, qd, ef, wk, wv, wek, wev)


def _attn_vbwd(res, ct):
    xs, qd, ef, wk, wv, wek, wev = res
    dlogits, dv = ct
    e = xs.shape[0]
    te, nb = _edge_grid(e)
    dxs, dqd, def_ = pl.pallas_call(
        _attn_bwd_body,
        grid=(nb,),
        in_specs=[
            _espec(te, _C), _espec(te, _DATTN), _espec(te, _C),
            _wspec((_C, _DATTN)), _wspec((_C, _DATTN)),
            _wspec((_DATTN, _C)), _wspec((_DATTN, _C)),
            _wspec((_DATTN, _C)), _wspec((_DATTN, _C)),
            _espec(te, _HEADS), _espec(te, _DATTN),
        ],
        out_specs=[_espec(te, _C), _espec(te, _DATTN), _espec(te, _C)],
        out_shape=[
            jax.ShapeDtypeStruct((e, _C), jnp.float32),
            jax.ShapeDtypeStruct((e, _DATTN), jnp.float32),
            jax.ShapeDtypeStruct((e, _C), jnp.float32),
        ],
    )(xs, qd, ef, wk, wek, wv.T, wev.T, wk.T, wek.T, dlogits, dv)
    return (dxs, dqd, def_, jnp.zeros_like(wk), jnp.zeros_like(wv),
            jnp.zeros_like(wek), jnp.zeros_like(wev))


_attn.defvjp(_attn_vfwd, _attn_vbwd)


def _soft_fwd_body(scale_ref, logits_ref, md_ref, v_ref, num_ref, numv_ref):
    e = jnp.exp(logits_ref[...] - md_ref[...])
    num = scale_ref[...] * e
    num_ref[...] = num
    numv_ref[...] = jnp.dot(num, _m8x16(), preferred_element_type=jnp.float32) * v_ref[...]


def _soft_bwd_body(scale_ref, logits_ref, md_ref, v_ref, dnum_ref, dnumv_ref,
                   dscale_ref, dlogits_ref, dv_ref):
    e = jnp.exp(logits_ref[...] - md_ref[...])
    num = scale_ref[...] * e
    dnumv = dnumv_ref[...]
    dv_ref[...] = jnp.dot(num, _m8x16(), preferred_element_type=jnp.float32) * dnumv
    dnum_tot = dnum_ref[...] + jnp.dot(
        dnumv * v_ref[...], _m16x8(), preferred_element_type=jnp.float32)
    dlogits_ref[...] = dnum_tot * num
    dscale_ref[...] = jnp.sum(dnum_tot * e, axis=1, keepdims=True)


@jax.custom_vjp
def _soft(scale, logits, md, v):
    e = logits.shape[0]
    te, nb = _edge_grid(e)
    return pl.pallas_call(
        _soft_fwd_body,
        grid=(nb,),
        in_specs=[_espec(te, 1), _espec(te, _HEADS), _espec(te, _HEADS),
                  _espec(te, _DATTN)],
        out_specs=[_espec(te, _HEADS), _espec(te, _DATTN)],
        out_shape=[
            jax.ShapeDtypeStruct((e, _HEADS), jnp.float32),
            jax.ShapeDtypeStruct((e, _DATTN), jnp.float32),
        ],
    )(scale, logits, md, v)


def _soft_vfwd(scale, logits, md, v):
    return _soft(scale, logits, md, v), (scale, logits, md, v)


def _soft_vbwd(res, ct):
    scale, logits, md, v = res
    dnum, dnumv = ct
    e = logits.shape[0]
    te, nb = _edge_grid(e)
    dscale, dlogits, dv = pl.pallas_call(
        _soft_bwd_body,
        grid=(nb,),
        in_specs=[_espec(te, 1), _espec(te, _HEADS), _espec(te, _HEADS),
                  _espec(te, _DATTN), _espec(te, _HEADS), _espec(te, _DATTN)],
        out_specs=[_espec(te, 1), _espec(te, _HEADS), _espec(te, _DATTN)],
        out_shape=[
            jax.ShapeDtypeStruct((e, 1), jnp.float32),
            jax.ShapeDtypeStruct((e, _HEADS), jnp.float32),
            jax.ShapeDtypeStruct((e, _DATTN), jnp.float32),
        ],
    )(scale, logits, md, v, dnum, dnumv)
    return (dscale, dlogits, -dlogits, dv)


_soft.defvjp(_soft_vfwd, _soft_vbwd)


def _final_fwd_body(xs_ref, ef_ref, scale_ref, won_ref, woe_ref, w1_ref,
                    b1_ref, w2_ref, b2_ref, out_ref):
    fe = jnp.dot(xs_ref[...], won_ref[...], preferred_element_type=jnp.float32) * jnp.dot(
        ef_ref[...], woe_ref[...], preferred_element_type=jnp.float32)
    z1 = jnp.dot(fe, w1_ref[...], preferred_element_type=jnp.float32) + b1_ref[...]
    h = z1 / (1.0 + jnp.exp(-z1))
    z2 = jnp.dot(h, w2_ref[...], preferred_element_type=jnp.float32) + b2_ref[...]
    s = jnp.sum(z2 * scale_ref[...], keepdims=True)

    @pl.when(pl.program_id(0) == 0)
    def _():
        out_ref[...] = jnp.zeros((1, 1), jnp.float32)

    out_ref[...] += s


def _final_bwd_body(xs_ref, ef_ref, scale_ref, won_ref, woe_ref, w1_ref,
                    b1_ref, w2_ref, b2_ref, w1t_ref, wont_ref, woet_ref,
                    g_ref, dxs_ref, def_ref, dscale_ref):
    a = jnp.dot(xs_ref[...], won_ref[...], preferred_element_type=jnp.float32)
    b = jnp.dot(ef_ref[...], woe_ref[...], preferred_element_type=jnp.float32)
    fe = a * b
    z1 = jnp.dot(fe, w1_ref[...], preferred_element_type=jnp.float32) + b1_ref[...]
    sig = 1.0 / (1.0 + jnp.exp(-z1))
    h = z1 * sig
    z2 = jnp.dot(h, w2_ref[...], preferred_element_type=jnp.float32) + b2_ref[...]
    g = g_ref[...]
    de = g * scale_ref[...]
    dscale_ref[...] = g * z2
    dh = de * jnp.transpose(w2_ref[...])
    dz1 = dh * (sig * (1.0 + z1 * (1.0 - sig)))
    dfe = jnp.dot(dz1, w1t_ref[...], preferred_element_type=jnp.float32)
    dxs_ref[...] = jnp.dot(dfe * b, wont_ref[...], preferred_element_type=jnp.float32)
    def_ref[...] = jnp.dot(dfe * a, woet_ref[...], preferred_element_type=jnp.float32)


@jax.custom_vjp
def _final(xs, ef, scale, won, woe, w1, b1, w2, b2):
    e = xs.shape[0]
    te, nb = _edge_grid(e)
    out = pl.pallas_call(
        _final_fwd_body,
        grid=(nb,),
        in_specs=[
            _espec(te, _C), _espec(te, _C), _espec(te, 1),
            _wspec((_C, _OUTC)), _wspec((_C, _OUTC)),
            _wspec((_OUTC, _OUTC)), _wspec((1, _OUTC)),
            _wspec((_OUTC, 1)), _wspec((1, 1)),
        ],
        out_specs=pl.BlockSpec((1, 1), lambda i: (0, 0)),
        out_shape=jax.ShapeDtypeStruct((1, 1), jnp.float32),
    )(xs, ef, scale, won, woe, w1, b1[None, :], w2, b2[None, :])
    return out[0, 0]


def _final_vfwd(xs, ef, scale, won, woe, w1, b1, w2, b2):
    return _final(xs, ef, scale, won, woe, w1, b1, w2, b2), (
        xs, ef, scale, won, woe, w1, b1, w2, b2)


def _final_vbwd(res, g):
    xs, ef, scale, won, woe, w1, b1, w2, b2 = res
    e = xs.shape[0]
    te, nb = _edge_grid(e)
    dxs, def_, dscale = pl.pallas_call(
        _final_bwd_body,
        grid=(nb,),
        in_specs=[
            _espec(te, _C), _espec(te, _C), _espec(te, 1),
            _wspec((_C, _OUTC)), _wspec((_C, _OUTC)),
            _wspec((_OUTC, _OUTC)), _wspec((1, _OUTC)),
            _wspec((_OUTC, 1)), _wspec((1, 1)),
            _wspec((_OUTC, _OUTC)), _wspec((_OUTC, _C)), _wspec((_OUTC, _C)),
            _wspec((1, 1)),
        ],
        out_specs=[_espec(te, _C), _espec(te, _C), _espec(te, 1)],
        out_shape=[
            jax.ShapeDtypeStruct((e, _C), jnp.float32),
            jax.ShapeDtypeStruct((e, _C), jnp.float32),
            jax.ShapeDtypeStruct((e, 1), jnp.float32),
        ],
    )(xs, ef, scale, won, woe, w1, b1[None, :], w2, b2[None, :],
      w1.T, won.T, woe.T, jnp.reshape(g, (1, 1)).astype(jnp.float32))
    return (dxs, def_, dscale, jnp.zeros_like(won), jnp.zeros_like(woe),
            jnp.zeros_like(w1), jnp.zeros_like(b1), jnp.zeros_like(w2),
            jnp.zeros_like(b2))


_final.defvjp(_final_vfwd, _final_vbwd)


def _energy_impl(pos, species, src, dst, params):
    n = pos.shape[0]
    rel = jnp.take(pos, dst, axis=0) - jnp.take(pos, src, axis=0)
    scale, ef = _geom(rel)
    x = jnp.take(params['embedding'], species - 1, axis=0)
    for lp in params['layers']:
        q = x @ lp['Wq']
        xs = jnp.take(x, src, axis=0)
        qd = jnp.take(q, dst, axis=0)
        logits, v = _attn(xs, qd, ef, lp['Wk'], lp['Wv'], lp['Wek'], lp['Wev'])
        m = jax.ops.segment_max(logits, dst, num_segments=n)
        m0 = jnp.where(jnp.isfinite(m), m, 0.0)
        md = jnp.take(m0, dst, axis=0)
        num, numv = _soft(scale, logits, md, v)
        den = jax.ops.segment_sum(num, dst, num_segments=n)
        sv = jax.ops.segment_sum(numv, dst, num_segments=n)
        agg = (sv.reshape(n, _HEADS, _HD)
               / (den[..., None] + 1e-9)).reshape(n, _DATTN)
        x = agg @ lp['Wo'] + x @ lp['Wskip']
        mu = jnp.mean(x, axis=-1, keepdims=True)
        var = jnp.var(x, axis=-1, keepdims=True)
        x = (x - mu) / jnp.sqrt(var + 1e-5) * lp['gamma'] + lp['beta']
    xs = jnp.take(x, src, axis=0)
    return _final(xs, ef, scale, params['Won'], params['Woe'],
                  params['mlp_w1'], params['mlp_b1'],
                  params['mlp_w2'], params['mlp_b2'])


def kernel(pos, species, edge_index, params):
    src = edge_index[0]
    dst = edge_index[1]

    def efn(p):
        return _energy_impl(p, species, src, dst, params)

    energy, dpos = jax.value_and_grad(efn)(pos)
    return energy, -dpos


# speedup vs baseline: 11.8699x; 2.4994x over previous
import functools

import jax
import jax.numpy as jnp
import numpy as np
from jax import lax
from jax.experimental import pallas as pl
from jax.experimental.pallas import tpu as pltpu
from jax.experimental.pallas import tpu_sc as plsc

_CUTOFF = 4.5
_C = 32
_NUM_DEGREES = 3
_OUTC = _C * _NUM_DEGREES
_HEADS = 8
_DATTN = 16
_HD = _DATTN // _HEADS
_INV_SQRT_HD = 1.0 / np.sqrt(_HD)

_DX = _CUTOFF / (_C - 2)


def _m16x8():
    row = jax.lax.broadcasted_iota(jnp.int32, (_DATTN, _HEADS), 0) // _HD
    col = jax.lax.broadcasted_iota(jnp.int32, (_DATTN, _HEADS), 1)
    return (row == col).astype(jnp.float32)


def _m8x16():
    row = jax.lax.broadcasted_iota(jnp.int32, (_HEADS, _DATTN), 0)
    col = jax.lax.broadcasted_iota(jnp.int32, (_HEADS, _DATTN), 1) // _HD
    return (row == col).astype(jnp.float32)


def _centers():
    return jax.lax.broadcasted_iota(
        jnp.int32, (1, _C - 1), 1).astype(jnp.float32) * _DX


def _edge_tile(e):
    for te in (4000, 2048, 1024, 512, 256, 128, 64, 32, 16, 8):
        if e % te == 0:
            return te
    return e


def _edge_grid(e):
    te = _edge_tile(e)
    return te, e // te


def _espec(te, width):
    return pl.BlockSpec((te, width), lambda i: (i, 0))


def _wspec(shape):
    nd = len(shape)
    return pl.BlockSpec(shape, lambda i: (0,) * nd)


_NC = 2
_NS = 16
_NW = _NC * _NS
_CH = 80


def _acc_rows(n):
    return -(-n // (_NS * 8)) * 8


def _sc_shapes_ok(e, d, n):
    n_pad = _NS * _acc_rows(n)
    return (
        e % (_NW * _CH) == 0
        and d in (8, 16, 24, 32)
        and n_pad * d <= 1_210_000
    )


def _sc_gather_impl(table, idx):
    n, d = table.shape
    e = idx.shape[0]
    per_w = e // _NW
    nch = per_w // _CH
    idx3 = idx.reshape(_NW, nch, _CH)
    mesh = plsc.VectorSubcoreMesh(core_axis_name="c", subcore_axis_name="s")

    @functools.partial(
        pl.kernel,
        mesh=mesh,
        compiler_params=pltpu.CompilerParams(use_tc_tiling_on_sc=False),
        out_type=jax.ShapeDtypeStruct((e, d), jnp.float32),
        scratch_types=[
            pltpu.VMEM((nch, _CH), jnp.int32),
            pltpu.VMEM((_CH, d), jnp.float32),
            pltpu.VMEM((_CH, d), jnp.float32),
            pltpu.SemaphoreType.DMA,
            pltpu.SemaphoreType.DMA,
        ],
    )
    def k(table_hbm, idx_hbm, out_hbm, idx_v, buf0, buf1, sem0, sem1):
        wid = lax.axis_index("s") * _NC + lax.axis_index("c")
        base = wid * per_w
        pltpu.sync_copy(idx_hbm.at[wid], idx_v)

        def body(i, carry):
            j0 = 2 * i
            j1 = j0 + 1
            h0 = pltpu.async_copy(table_hbm.at[idx_v.at[j0]], buf0, sem0)
            h1 = pltpu.async_copy(table_hbm.at[idx_v.at[j1]], buf1, sem1)
            h0.wait()
            pltpu.sync_copy(buf0, out_hbm.at[pl.ds(base + j0 * _CH, _CH)])
            h1.wait()
            pltpu.sync_copy(buf1, out_hbm.at[pl.ds(base + j1 * _CH, _CH)])
            return carry

        lax.fori_loop(0, nch // 2, body, 0)
        if nch % 2:
            jl = nch - 1
            pltpu.async_copy(table_hbm.at[idx_v.at[jl]], buf0, sem0).wait()
            pltpu.sync_copy(buf0, out_hbm.at[pl.ds(base + jl * _CH, _CH)])

    return k(table, idx3)


def _sc_segsum_impl(data, idx, n):
    e, d = data.shape
    per_w = e // _NW
    nch = per_w // _CH
    rows_t = _acc_rows(n)
    n_pad = _NS * rows_t
    idx3 = idx.reshape(_NW, nch, _CH)
    zeros = jnp.zeros((rows_t, d), jnp.float32)
    mesh = plsc.VectorSubcoreMesh(core_axis_name="c", subcore_axis_name="s")

    @functools.partial(
        pl.kernel,
        mesh=mesh,
        compiler_params=pltpu.CompilerParams(use_tc_tiling_on_sc=False),
        out_type=jax.ShapeDtypeStruct((_NC, n_pad, d), jnp.float32),
        scratch_types=[
            pltpu.VMEM((nch, _CH), jnp.int32),
            pltpu.VMEM((_CH, d), jnp.float32),
            pltpu.VMEM((_CH, d), jnp.float32),
            pltpu.VMEM_SHARED((n_pad, d), jnp.float32),
            pltpu.SemaphoreType.DMA,
            pltpu.SemaphoreType.DMA,
        ],
    )
    def k(data_hbm, idx_hbm, z_hbm, out_hbm, idx_v, buf0, buf1, acc,
          sem0, sem1):
        cid = lax.axis_index("c")
        sid = lax.axis_index("s")
        wid = sid * _NC + cid
        base = wid * per_w
        pltpu.sync_copy(idx_hbm.at[wid], idx_v)
        r0 = sid * rows_t
        pltpu.sync_copy(z_hbm, acc.at[pl.ds(r0, rows_t)])
        plsc.subcore_barrier()

        def body(i, carry):
            j0 = 2 * i
            j1 = j0 + 1
            h0 = pltpu.async_copy(
                data_hbm.at[pl.ds(base + j0 * _CH, _CH)], buf0, sem0)
            h1 = pltpu.async_copy(
                data_hbm.at[pl.ds(base + j1 * _CH, _CH)], buf1, sem1)
            h0.wait()
            pltpu.sync_copy(buf0, acc.at[idx_v.at[j0]], add=True)
            h1.wait()
            pltpu.sync_copy(buf1, acc.at[idx_v.at[j1]], add=True)
            return carry

        lax.fori_loop(0, nch // 2, body, 0)
        if nch % 2:
            jl = nch - 1
            pltpu.async_copy(
                data_hbm.at[pl.ds(base + jl * _CH, _CH)], buf0, sem0).wait()
            pltpu.sync_copy(buf0, acc.at[idx_v.at[jl]], add=True)
        plsc.subcore_barrier()
        pltpu.sync_copy(acc.at[pl.ds(r0, rows_t)],
                        out_hbm.at[cid, pl.ds(r0, rows_t)])

    out = k(data, idx3, zeros)
    return (out[0] + out[1])[:n]


def _gather_dispatch(table, idx):
    if _sc_shapes_ok(idx.shape[0], table.shape[1], table.shape[0]):
        return _sc_gather_impl(table, idx)
    return jnp.take(table, idx, axis=0)


def _segsum_dispatch(data, idx, n):
    e, d = data.shape
    if _sc_shapes_ok(e, d, n):
        return _sc_segsum_impl(data, idx, n)
    if d % 16 == 0 and _sc_shapes_ok(e, 16, n):
        return jnp.concatenate(
            [_sc_segsum_impl(data[:, j:j + 16], idx, n)
             for j in range(0, d, 16)], axis=1)
    return jax.ops.segment_sum(data, idx, num_segments=n)


def _int_zero(idx):
    return np.zeros(idx.shape, dtype=jax.dtypes.float0)


@functools.lru_cache(maxsize=None)
def _gather_fn(n_rows):
    @jax.custom_vjp
    def g(table, idx):
        return _gather_dispatch(table, idx)

    def g_fwd(table, idx):
        return g(table, idx), idx

    def g_bwd(idx, ct):
        return (_segsum_dispatch(ct, idx, n_rows), _int_zero(idx))

    g.defvjp(g_fwd, g_bwd)
    return g


@functools.lru_cache(maxsize=None)
def _segsum_fn(n_rows):
    @jax.custom_vjp
    def s(data, idx):
        return _segsum_dispatch(data, idx, n_rows)

    def s_fwd(data, idx):
        return s(data, idx), idx

    def s_bwd(idx, ct):
        return (_gather_dispatch(ct, idx), _int_zero(idx))

    s.defvjp(s_fwd, s_bwd)
    return s


def _geom_fwd_body(rel_ref, scale_ref, ef_ref):
    rel = rel_ref[...]
    d2 = jnp.sum(rel * rel, axis=1, keepdims=True) + 1e-12
    d = jnp.sqrt(d2)
    inside = d < _CUTOFF
    xc = jnp.where(inside, d / _CUTOFF, 0.5)
    bump = jnp.exp(1.0 - 1.0 / (1.0 - xc * xc))
    smooth = jnp.exp(-1.0 / jnp.maximum(d, 1e-9))
    scale_ref[...] = jnp.where(inside, bump * smooth, 0.0)
    rb = jnp.exp(-2.0 * (d - _centers()) ** 2 / _DX**2)
    r_soft = jnp.sqrt(d2 + 1.0) - 1.0
    ef_ref[...] = jnp.concatenate([rb, r_soft], axis=1)


def _geom_bwd_body(rel_ref, dscale_ref, def_ref, drel_ref):
    rel = rel_ref[...]
    d2 = jnp.sum(rel * rel, axis=1, keepdims=True) + 1e-12
    d = jnp.sqrt(d2)
    inside = d < _CUTOFF
    xc = jnp.where(inside, d / _CUTOFF, 0.5)
    bump = jnp.exp(1.0 - 1.0 / (1.0 - xc * xc))
    smooth = jnp.exp(-1.0 / jnp.maximum(d, 1e-9))
    one_m = 1.0 - xc * xc
    g_scale = jnp.where(
        inside,
        bump * smooth * (1.0 / d2 - 2.0 * xc / (_CUTOFF * one_m * one_m)),
        0.0,
    )
    rb = jnp.exp(-2.0 * (d - _centers()) ** 2 / _DX**2)
    g_rb = rb * (-4.0 * (d - _centers()) / _DX**2)
    g_rsoft = d / jnp.sqrt(d2 + 1.0)
    def_ = def_ref[...]
    dd = (
        dscale_ref[...] * g_scale
        + jnp.sum(def_[:, : _C - 1] * g_rb, axis=1, keepdims=True)
        + def_[:, _C - 1 : _C] * g_rsoft
    )
    drel_ref[...] = rel * (dd / d)


@jax.custom_vjp
def _geom(rel):
    te, nb = _edge_grid(rel.shape[0])
    return pl.pallas_call(
        _geom_fwd_body,
        grid=(nb,),
        in_specs=[_espec(te, 8)],
        out_specs=[_espec(te, 1), _espec(te, _C)],
        out_shape=[
            jax.ShapeDtypeStruct((rel.shape[0], 1), jnp.float32),
            jax.ShapeDtypeStruct((rel.shape[0], _C), jnp.float32),
        ],
    )(rel)


def _geom_vfwd(rel):
    return _geom(rel), (rel,)


def _geom_vbwd(res, ct):
    (rel,) = res
    dscale, def_ = ct
    te, nb = _edge_grid(rel.shape[0])
    drel = pl.pallas_call(
        _geom_bwd_body,
        grid=(nb,),
        in_specs=[_espec(te, 8), _espec(te, 1), _espec(te, _C)],
        out_specs=_espec(te, 8),
        out_shape=jax.ShapeDtypeStruct(rel.shape, jnp.float32),
    )(rel, dscale, def_)
    return (drel,)


_geom.defvjp(_geom_vfwd, _geom_vbwd)


def _attn_fwd_body(xs_ref, qd_ref, ef_ref, wk_ref, wv_ref, wek_ref, wev_ref,
                   logits_ref, v_ref):
    xs = xs_ref[...]
    ef = ef_ref[...]
    k = jnp.dot(xs, wk_ref[...], preferred_element_type=jnp.float32) + jnp.dot(
        ef, wek_ref[...], preferred_element_type=jnp.float32)
    v = jnp.dot(xs, wv_ref[...], preferred_element_type=jnp.float32) + jnp.dot(
        ef, wev_ref[...], preferred_element_type=jnp.float32)
    logits_ref[...] = jnp.dot(
        qd_ref[...] * k, _m16x8(), preferred_element_type=jnp.float32) * _INV_SQRT_HD
    v_ref[...] = v


def _attn_bwd_body(xs_ref, qd_ref, ef_ref, wk_ref, wek_ref,
                   wvt_ref, wevt_ref, wkt_ref, wekt_ref,
                   dlogits_ref, dv_ref,
                   dxs_ref, dqd_ref, def_ref):
    xs = xs_ref[...]
    ef = ef_ref[...]
    k = jnp.dot(xs, wk_ref[...], preferred_element_type=jnp.float32) + jnp.dot(
        ef, wek_ref[...], preferred_element_type=jnp.float32)
    dl16 = jnp.dot(dlogits_ref[...] * _INV_SQRT_HD, _m8x16(),
                   preferred_element_type=jnp.float32)
    dqd_ref[...] = dl16 * k
    dk = dl16 * qd_ref[...]
    dv = dv_ref[...]
    dxs_ref[...] = jnp.dot(dk, wkt_ref[...], preferred_element_type=jnp.float32) + jnp.dot(
        dv, wvt_ref[...], preferred_element_type=jnp.float32)
    def_ref[...] = jnp.dot(dk, wekt_ref[...], preferred_element_type=jnp.float32) + jnp.dot(
        dv, wevt_ref[...], preferred_element_type=jnp.float32)


@jax.custom_vjp
def _attn(xs, qd, ef, wk, wv, wek, wev):
    e = xs.shape[0]
    te, nb = _edge_grid(e)
    return pl.pallas_call(
        _attn_fwd_body,
        grid=(nb,),
        in_specs=[
            _espec(te, _C), _espec(te, _DATTN), _espec(te, _C),
            _wspec((_C, _DATTN)), _wspec((_C, _DATTN)),
            _wspec((_C, _DATTN)), _wspec((_C, _DATTN)),
        ],
        out_specs=[_espec(te, _HEADS), _espec(te, _DATTN)],
        out_shape=[
            jax.ShapeDtypeStruct((e, _HEADS), jnp.float32),
            jax.ShapeDtypeStruct((e, _DATTN), jnp.float32),
        ],
    )(xs, qd, ef, wk, wv, wek, wev)


def _attn_vfwd(xs, qd, ef, wk, wv, wek, wev):
    return _attn(xs, qd, ef, wk, wv, wek, wev), (xs, qd, ef, wk, wv, wek, wev)


def _attn_vbwd(res, ct):
    xs, qd, ef, wk, wv, wek, wev = res
    dlogits, dv = ct
    e = xs.shape[0]
    te, nb = _edge_grid(e)
    dxs, dqd, def_ = pl.pallas_call(
        _attn_bwd_body,
        grid=(nb,),
        in_specs=[
            _espec(te, _C), _espec(te, _DATTN), _espec(te, _C),
            _wspec((_C, _DATTN)), _wspec((_C, _DATTN)),
            _wspec((_DATTN, _C)), _wspec((_DATTN, _C)),
            _wspec((_DATTN, _C)), _wspec((_DATTN, _C)),
            _espec(te, _HEADS), _espec(te, _DATTN),
        ],
        out_specs=[_espec(te, _C), _espec(te, _DATTN), _espec(te, _C)],
        out_shape=[
            jax.ShapeDtypeStruct((e, _C), jnp.float32),
            jax.ShapeDtypeStruct((e, _DATTN), jnp.float32),
            jax.ShapeDtypeStruct((e, _C), jnp.float32),
        ],
    )(xs, qd, ef, wk, wek, wv.T, wev.T, wk.T, wek.T, dlogits, dv)
    return (dxs, dqd, def_, jnp.zeros_like(wk), jnp.zeros_like(wv),
            jnp.zeros_like(wek), jnp.zeros_like(wev))


_attn.defvjp(_attn_vfwd, _attn_vbwd)


def _soft_fwd_body(scale_ref, logits_ref, md_ref, v_ref, numw_ref):
    e = jnp.exp(logits_ref[...] - md_ref[...])
    num = scale_ref[...] * e
    numv = jnp.dot(num, _m8x16(), preferred_element_type=jnp.float32) * v_ref[...]
    numw_ref[...] = jnp.concatenate([num, numv], axis=1)


def _soft_bwd_body(scale_ref, logits_ref, md_ref, v_ref, ct_ref,
                   dscale_ref, dlogits_ref, dv_ref):
    e = jnp.exp(logits_ref[...] - md_ref[...])
    num = scale_ref[...] * e
    ct = ct_ref[...]
    dnum = ct[:, :_HEADS]
    dnumv = ct[:, _HEADS:]
    dv_ref[...] = jnp.dot(num, _m8x16(), preferred_element_type=jnp.float32) * dnumv
    dnum_tot = dnum + jnp.dot(
        dnumv * v_ref[...], _m16x8(), preferred_element_type=jnp.float32)
    dlogits_ref[...] = dnum_tot * num
    dscale_ref[...] = jnp.sum(dnum_tot * e, axis=1, keepdims=True)


@jax.custom_vjp
def _soft(scale, logits, md, v):
    e = logits.shape[0]
    te, nb = _edge_grid(e)
    return pl.pallas_call(
        _soft_fwd_body,
        grid=(nb,),
        in_specs=[_espec(te, 1), _espec(te, _HEADS), _espec(te, _HEADS),
                  _espec(te, _DATTN)],
        out_specs=_espec(te, _HEADS + _DATTN),
        out_shape=jax.ShapeDtypeStruct((e, _HEADS + _DATTN), jnp.float32),
    )(scale, logits, md, v)


def _soft_vfwd(scale, logits, md, v):
    return _soft(scale, logits, md, v), (scale, logits, md, v)


def _soft_vbwd(res, ct):
    scale, logits, md, v = res
    e = logits.shape[0]
    te, nb = _edge_grid(e)
    dscale, dlogits, dv = pl.pallas_call(
        _soft_bwd_body,
        grid=(nb,),
        in_specs=[_espec(te, 1), _espec(te, _HEADS), _espec(te, _HEADS),
                  _espec(te, _DATTN), _espec(te, _HEADS + _DATTN)],
        out_specs=[_espec(te, 1), _espec(te, _HEADS), _espec(te, _DATTN)],
        out_shape=[
            jax.ShapeDtypeStruct((e, 1), jnp.float32),
            jax.ShapeDtypeStruct((e, _HEADS), jnp.float32),
            jax.ShapeDtypeStruct((e, _DATTN), jnp.float32),
        ],
    )(scale, logits, md, v, ct)
    return (dscale, dlogits, jnp.zeros_like(md), dv)


_soft.defvjp(_soft_vfwd, _soft_vbwd)


def _final_fwd_body(xs_ref, ef_ref, scale_ref, won_ref, woe_ref, w1_ref,
                    b1_ref, w2_ref, b2_ref, out_ref):
    fe = jnp.dot(xs_ref[...], won_ref[...], preferred_element_type=jnp.float32) * jnp.dot(
        ef_ref[...], woe_ref[...], preferred_element_type=jnp.float32)
    z1 = jnp.dot(fe, w1_ref[...], preferred_element_type=jnp.float32) + b1_ref[...]
    h = z1 / (1.0 + jnp.exp(-z1))
    z2 = jnp.dot(h, w2_ref[...], preferred_element_type=jnp.float32) + b2_ref[...]
    s = jnp.sum(z2 * scale_ref[...], keepdims=True)

    @pl.when(pl.program_id(0) == 0)
    def _():
        out_ref[...] = jnp.zeros((1, 1), jnp.float32)

    out_ref[...] += s


def _final_bwd_body(xs_ref, ef_ref, scale_ref, won_ref, woe_ref, w1_ref,
                    b1_ref, w2_ref, b2_ref, w1t_ref, wont_ref, woet_ref,
                    g_ref, dxs_ref, def_ref, dscale_ref):
    a = jnp.dot(xs_ref[...], won_ref[...], preferred_element_type=jnp.float32)
    b = jnp.dot(ef_ref[...], woe_ref[...], preferred_element_type=jnp.float32)
    fe = a * b
    z1 = jnp.dot(fe, w1_ref[...], preferred_element_type=jnp.float32) + b1_ref[...]
    sig = 1.0 / (1.0 + jnp.exp(-z1))
    h = z1 * sig
    z2 = jnp.dot(h, w2_ref[...], preferred_element_type=jnp.float32) + b2_ref[...]
    g = g_ref[...]
    de = g * scale_ref[...]
    dscale_ref[...] = g * z2
    dh = de * jnp.transpose(w2_ref[...])
    dz1 = dh * (sig * (1.0 + z1 * (1.0 - sig)))
    dfe = jnp.dot(dz1, w1t_ref[...], preferred_element_type=jnp.float32)
    dxs_ref[...] = jnp.dot(dfe * b, wont_ref[...], preferred_element_type=jnp.float32)
    def_ref[...] = jnp.dot(dfe * a, woet_ref[...], preferred_element_type=jnp.float32)


@jax.custom_vjp
def _final(xs, ef, scale, won, woe, w1, b1, w2, b2):
    e = xs.shape[0]
    te, nb = _edge_grid(e)
    out = pl.pallas_call(
        _final_fwd_body,
        grid=(nb,),
        in_specs=[
            _espec(te, _C), _espec(te, _C), _espec(te, 1),
            _wspec((_C, _OUTC)), _wspec((_C, _OUTC)),
            _wspec((_OUTC, _OUTC)), _wspec((1, _OUTC)),
            _wspec((_OUTC, 1)), _wspec((1, 1)),
        ],
        out_specs=pl.BlockSpec((1, 1), lambda i: (0, 0)),
        out_shape=jax.ShapeDtypeStruct((1, 1), jnp.float32),
    )(xs, ef, scale, won, woe, w1, b1[None, :], w2, b2[None, :])
    return out[0, 0]


def _final_vfwd(xs, ef, scale, won, woe, w1, b1, w2, b2):
    return _final(xs, ef, scale, won, woe, w1, b1, w2, b2), (
        xs, ef, scale, won, woe, w1, b1, w2, b2)


def _final_vbwd(res, g):
    xs, ef, scale, won, woe, w1, b1, w2, b2 = res
    e = xs.shape[0]
    te, nb = _edge_grid(e)
    dxs, def_, dscale = pl.pallas_call(
        _final_bwd_body,
        grid=(nb,),
        in_specs=[
            _espec(te, _C), _espec(te, _C), _espec(te, 1),
            _wspec((_C, _OUTC)), _wspec((_C, _OUTC)),
            _wspec((_OUTC, _OUTC)), _wspec((1, _OUTC)),
            _wspec((_OUTC, 1)), _wspec((1, 1)),
            _wspec((_OUTC, _OUTC)), _wspec((_OUTC, _C)), _wspec((_OUTC, _C)),
            _wspec((1, 1)),
        ],
        out_specs=[_espec(te, _C), _espec(te, _C), _espec(te, 1)],
        out_shape=[
            jax.ShapeDtypeStruct((e, _C), jnp.float32),
            jax.ShapeDtypeStruct((e, _C), jnp.float32),
            jax.ShapeDtypeStruct((e, 1), jnp.float32),
        ],
    )(xs, ef, scale, won, woe, w1, b1[None, :], w2, b2[None, :],
      w1.T, won.T, woe.T, jnp.reshape(g, (1, 1)).astype(jnp.float32))
    return (dxs, def_, dscale, jnp.zeros_like(won), jnp.zeros_like(woe),
            jnp.zeros_like(w1), jnp.zeros_like(b1), jnp.zeros_like(w2),
            jnp.zeros_like(b2))


_final.defvjp(_final_vfwd, _final_vbwd)


def _energy_impl(pos, species, src, dst, params):
    n = pos.shape[0]
    gather = _gather_fn(n)
    segsum = _segsum_fn(n)
    pos4 = jnp.concatenate([pos, jnp.zeros((n, 5), jnp.float32)], axis=1)
    rel = gather(pos4, dst) - gather(pos4, src)
    scale, ef = _geom(rel)
    x = jnp.take(params['embedding'], species - 1, axis=0)
    for lp in params['layers']:
        q = x @ lp['Wq']
        xs = gather(x, src)
        qd = gather(q, dst)
        logits, v = _attn(xs, qd, ef, lp['Wk'], lp['Wv'], lp['Wek'], lp['Wev'])
        m = jax.ops.segment_max(jax.lax.stop_gradient(logits), dst,
                                num_segments=n)
        m0 = jnp.where(jnp.isfinite(m), m, 0.0)
        md = jax.lax.stop_gradient(gather(m0, dst))
        numw = _soft(scale, logits, md, v)
        segw = segsum(numw, dst)
        den = segw[:, :_HEADS]
        sv = segw[:, _HEADS:]
        agg = (sv.reshape(n, _HEADS, _HD)
               / (den[..., None] + 1e-9)).reshape(n, _DATTN)
        x = agg @ lp['Wo'] + x @ lp['Wskip']
        mu = jnp.mean(x, axis=-1, keepdims=True)
        var = jnp.var(x, axis=-1, keepdims=True)
        x = (x - mu) / jnp.sqrt(var + 1e-5) * lp['gamma'] + lp['beta']
    xs = gather(x, src)
    return _final(xs, ef, scale, params['Won'], params['Woe'],
                  params['mlp_w1'], params['mlp_b1'],
                  params['mlp_w2'], params['mlp_b2'])


def kernel(pos, species, edge_index, params):
    src = edge_index[0]
    dst = edge_index[1]

    def efn(p):
        return _energy_impl(p, species, src, dst, params)

    energy, dpos = jax.value_and_grad(efn)(pos)
    return energy, -dpos


# speedup vs baseline: 14.5467x; 1.2255x over previous
import functools

import jax
import jax.numpy as jnp
import numpy as np
from jax import lax
from jax.experimental import pallas as pl
from jax.experimental.pallas import tpu as pltpu
from jax.experimental.pallas import tpu_sc as plsc

_CUTOFF = 4.5
_C = 32
_NUM_DEGREES = 3
_OUTC = _C * _NUM_DEGREES
_HEADS = 8
_DATTN = 16
_HD = _DATTN // _HEADS
_INV_SQRT_HD = 1.0 / np.sqrt(_HD)

_DX = _CUTOFF / (_C - 2)


def _m16x8():
    row = jax.lax.broadcasted_iota(jnp.int32, (_DATTN, _HEADS), 0) // _HD
    col = jax.lax.broadcasted_iota(jnp.int32, (_DATTN, _HEADS), 1)
    return (row == col).astype(jnp.float32)


def _m8x16():
    row = jax.lax.broadcasted_iota(jnp.int32, (_HEADS, _DATTN), 0)
    col = jax.lax.broadcasted_iota(jnp.int32, (_HEADS, _DATTN), 1) // _HD
    return (row == col).astype(jnp.float32)


def _centers():
    return jax.lax.broadcasted_iota(
        jnp.int32, (1, _C - 1), 1).astype(jnp.float32) * _DX


def _edge_tile(e):
    for te in (4000, 2048, 1024, 512, 256, 128, 64, 32, 16, 8):
        if e % te == 0:
            return te
    return e


def _edge_grid(e):
    te = _edge_tile(e)
    return te, e // te


def _espec(te, width):
    return pl.BlockSpec((te, width), lambda i: (i, 0))


def _wspec(shape):
    nd = len(shape)
    return pl.BlockSpec(shape, lambda i: (0,) * nd)


_NC = 2
_NS = 16
_NW = _NC * _NS
_CH = 80


def _acc_rows(n):
    return -(-n // (_NS * 8)) * 8


def _sc_shapes_ok(e, d, n):
    n_pad = _NS * _acc_rows(n)
    return (
        e % (_NW * _CH) == 0
        and d in (8, 16, 24, 32)
        and n_pad * d <= 1_210_000
    )


def _sc_gather_impl(table, idx):
    n, d = table.shape
    e = idx.shape[0]
    per_w = e // _NW
    nch = per_w // _CH
    idx3 = idx.reshape(_NW, nch, _CH)
    mesh = plsc.VectorSubcoreMesh(core_axis_name="c", subcore_axis_name="s")

    @functools.partial(
        pl.kernel,
        mesh=mesh,
        compiler_params=pltpu.CompilerParams(use_tc_tiling_on_sc=False),
        out_type=jax.ShapeDtypeStruct((e, d), jnp.float32),
        scratch_types=[
            pltpu.VMEM((nch, _CH), jnp.int32),
            pltpu.VMEM((_CH, d), jnp.float32),
            pltpu.VMEM((_CH, d), jnp.float32),
            pltpu.SemaphoreType.DMA,
            pltpu.SemaphoreType.DMA,
        ],
    )
    def k(table_hbm, idx_hbm, out_hbm, idx_v, buf0, buf1, sem0, sem1):
        wid = lax.axis_index("s") * _NC + lax.axis_index("c")
        base = wid * per_w
        pltpu.sync_copy(idx_hbm.at[wid], idx_v)

        def body(i, carry):
            j0 = 2 * i
            j1 = j0 + 1
            h0 = pltpu.async_copy(table_hbm.at[idx_v.at[j0]], buf0, sem0)
            h1 = pltpu.async_copy(table_hbm.at[idx_v.at[j1]], buf1, sem1)
            h0.wait()
            pltpu.sync_copy(buf0, out_hbm.at[pl.ds(base + j0 * _CH, _CH)])
            h1.wait()
            pltpu.sync_copy(buf1, out_hbm.at[pl.ds(base + j1 * _CH, _CH)])
            return carry

        lax.fori_loop(0, nch // 2, body, 0)
        if nch % 2:
            jl = nch - 1
            pltpu.async_copy(table_hbm.at[idx_v.at[jl]], buf0, sem0).wait()
            pltpu.sync_copy(buf0, out_hbm.at[pl.ds(base + jl * _CH, _CH)])

    return k(table, idx3)


def _sc_segsum_impl(data, idx, n):
    e, d = data.shape
    per_w = e // _NW
    nch = per_w // _CH
    rows_t = _acc_rows(n)
    n_pad = _NS * rows_t
    idx3 = idx.reshape(_NW, nch, _CH)
    zeros = jnp.zeros((rows_t, d), jnp.float32)
    mesh = plsc.VectorSubcoreMesh(core_axis_name="c", subcore_axis_name="s")

    @functools.partial(
        pl.kernel,
        mesh=mesh,
        compiler_params=pltpu.CompilerParams(use_tc_tiling_on_sc=False),
        out_type=jax.ShapeDtypeStruct((_NC, n_pad, d), jnp.float32),
        scratch_types=[
            pltpu.VMEM((nch, _CH), jnp.int32),
            pltpu.VMEM((_CH, d), jnp.float32),
            pltpu.VMEM((_CH, d), jnp.float32),
            pltpu.VMEM_SHARED((n_pad, d), jnp.float32),
            pltpu.SemaphoreType.DMA,
            pltpu.SemaphoreType.DMA,
        ],
    )
    def k(data_hbm, idx_hbm, z_hbm, out_hbm, idx_v, buf0, buf1, acc,
          sem0, sem1):
        cid = lax.axis_index("c")
        sid = lax.axis_index("s")
        wid = sid * _NC + cid
        base = wid * per_w
        pltpu.sync_copy(idx_hbm.at[wid], idx_v)
        r0 = sid * rows_t
        pltpu.sync_copy(z_hbm, acc.at[pl.ds(r0, rows_t)])
        plsc.subcore_barrier()

        def body(i, carry):
            j0 = 2 * i
            j1 = j0 + 1
            h0 = pltpu.async_copy(
                data_hbm.at[pl.ds(base + j0 * _CH, _CH)], buf0, sem0)
            h1 = pltpu.async_copy(
                data_hbm.at[pl.ds(base + j1 * _CH, _CH)], buf1, sem1)
            h0.wait()
            pltpu.sync_copy(buf0, acc.at[idx_v.at[j0]], add=True)
            h1.wait()
            pltpu.sync_copy(buf1, acc.at[idx_v.at[j1]], add=True)
            return carry

        lax.fori_loop(0, nch // 2, body, 0)
        if nch % 2:
            jl = nch - 1
            pltpu.async_copy(
                data_hbm.at[pl.ds(base + jl * _CH, _CH)], buf0, sem0).wait()
            pltpu.sync_copy(buf0, acc.at[idx_v.at[jl]], add=True)
        plsc.subcore_barrier()
        pltpu.sync_copy(acc.at[pl.ds(r0, rows_t)],
                        out_hbm.at[cid, pl.ds(r0, rows_t)])

    out = k(data, idx3, zeros)
    return (out[0] + out[1])[:n]


def _gather_dispatch(table, idx):
    if _sc_shapes_ok(idx.shape[0], table.shape[1], table.shape[0]):
        return _sc_gather_impl(table, idx)
    return jnp.take(table, idx, axis=0)


def _segsum_dispatch(data, idx, n):
    e, d = data.shape
    if _sc_shapes_ok(e, d, n):
        return _sc_segsum_impl(data, idx, n)
    if d % 16 == 0 and _sc_shapes_ok(e, 16, n):
        return jnp.concatenate(
            [_sc_segsum_impl(data[:, j:j + 16], idx, n)
             for j in range(0, d, 16)], axis=1)
    return jax.ops.segment_sum(data, idx, num_segments=n)


def _int_zero(idx):
    return np.zeros(idx.shape, dtype=jax.dtypes.float0)


@functools.lru_cache(maxsize=None)
def _gather_fn(n_rows):
    @jax.custom_vjp
    def g(table, idx):
        return _gather_dispatch(table, idx)

    def g_fwd(table, idx):
        return g(table, idx), idx

    def g_bwd(idx, ct):
        return (_segsum_dispatch(ct, idx, n_rows), _int_zero(idx))

    g.defvjp(g_fwd, g_bwd)
    return g


@functools.lru_cache(maxsize=None)
def _segsum_fn(n_rows):
    @jax.custom_vjp
    def s(data, idx):
        return _segsum_dispatch(data, idx, n_rows)

    def s_fwd(data, idx):
        return s(data, idx), idx

    def s_bwd(idx, ct):
        return (_gather_dispatch(ct, idx), _int_zero(idx))

    s.defvjp(s_fwd, s_bwd)
    return s


def _geom_fwd_body(rel_ref, scale_ref, ef_ref):
    rel = rel_ref[...]
    d2 = jnp.sum(rel * rel, axis=1, keepdims=True) + 1e-12
    d = jnp.sqrt(d2)
    inside = d < _CUTOFF
    xc = jnp.where(inside, d / _CUTOFF, 0.5)
    bump = jnp.exp(1.0 - 1.0 / (1.0 - xc * xc))
    smooth = jnp.exp(-1.0 / jnp.maximum(d, 1e-9))
    scale_ref[...] = jnp.where(inside, bump * smooth, 0.0)
    rb = jnp.exp(-2.0 * (d - _centers()) ** 2 / _DX**2)
    r_soft = jnp.sqrt(d2 + 1.0) - 1.0
    ef_ref[...] = jnp.concatenate([rb, r_soft], axis=1)


def _geom_bwd_body(rel_ref, dscale_ref, def_ref, drel_ref):
    rel = rel_ref[...]
    d2 = jnp.sum(rel * rel, axis=1, keepdims=True) + 1e-12
    d = jnp.sqrt(d2)
    inside = d < _CUTOFF
    xc = jnp.where(inside, d / _CUTOFF, 0.5)
    bump = jnp.exp(1.0 - 1.0 / (1.0 - xc * xc))
    smooth = jnp.exp(-1.0 / jnp.maximum(d, 1e-9))
    one_m = 1.0 - xc * xc
    g_scale = jnp.where(
        inside,
        bump * smooth * (1.0 / d2 - 2.0 * xc / (_CUTOFF * one_m * one_m)),
        0.0,
    )
    rb = jnp.exp(-2.0 * (d - _centers()) ** 2 / _DX**2)
    g_rb = rb * (-4.0 * (d - _centers()) / _DX**2)
    g_rsoft = d / jnp.sqrt(d2 + 1.0)
    def_ = def_ref[...]
    dd = (
        dscale_ref[...] * g_scale
        + jnp.sum(def_[:, : _C - 1] * g_rb, axis=1, keepdims=True)
        + def_[:, _C - 1 : _C] * g_rsoft
    )
    drel_ref[...] = rel * (dd / d)


@jax.custom_vjp
def _geom(rel):
    te, nb = _edge_grid(rel.shape[0])
    return pl.pallas_call(
        _geom_fwd_body,
        grid=(nb,),
        in_specs=[_espec(te, 8)],
        out_specs=[_espec(te, 1), _espec(te, _C)],
        out_shape=[
            jax.ShapeDtypeStruct((rel.shape[0], 1), jnp.float32),
            jax.ShapeDtypeStruct((rel.shape[0], _C), jnp.float32),
        ],
    )(rel)


def _geom_vfwd(rel):
    return _geom(rel), (rel,)


def _geom_vbwd(res, ct):
    (rel,) = res
    dscale, def_ = ct
    te, nb = _edge_grid(rel.shape[0])
    drel = pl.pallas_call(
        _geom_bwd_body,
        grid=(nb,),
        in_specs=[_espec(te, 8), _espec(te, 1), _espec(te, _C)],
        out_specs=_espec(te, 8),
        out_shape=jax.ShapeDtypeStruct(rel.shape, jnp.float32),
    )(rel, dscale, def_)
    return (drel,)


_geom.defvjp(_geom_vfwd, _geom_vbwd)


def _attn_fwd_body(xs_ref, qd_ref, ef_ref, wk_ref, wv_ref, wek_ref, wev_ref,
                   logits_ref, v_ref):
    xs = xs_ref[...]
    ef = ef_ref[...]
    k = jnp.dot(xs, wk_ref[...], preferred_element_type=jnp.float32) + jnp.dot(
        ef, wek_ref[...], preferred_element_type=jnp.float32)
    v = jnp.dot(xs, wv_ref[...], preferred_element_type=jnp.float32) + jnp.dot(
        ef, wev_ref[...], preferred_element_type=jnp.float32)
    logits_ref[...] = jnp.dot(
        qd_ref[...] * k, _m16x8(), preferred_element_type=jnp.float32) * _INV_SQRT_HD
    v_ref[...] = v


def _attn_bwd_body(xs_ref, qd_ref, ef_ref, wk_ref, wek_ref,
                   wvt_ref, wevt_ref, wkt_ref, wekt_ref,
                   dlogits_ref, dv_ref,
                   dxs_ref, dqd_ref, def_ref):
    xs = xs_ref[...]
    ef = ef_ref[...]
    k = jnp.dot(xs, wk_ref[...], preferred_element_type=jnp.float32) + jnp.dot(
        ef, wek_ref[...], preferred_element_type=jnp.float32)
    dl16 = jnp.dot(dlogits_ref[...] * _INV_SQRT_HD, _m8x16(),
                   preferred_element_type=jnp.float32)
    dqd_ref[...] = dl16 * k
    dk = dl16 * qd_ref[...]
    dv = dv_ref[...]
    dxs_ref[...] = jnp.dot(dk, wkt_ref[...], preferred_element_type=jnp.float32) + jnp.dot(
        dv, wvt_ref[...], preferred_element_type=jnp.float32)
    def_ref[...] = jnp.dot(dk, wekt_ref[...], preferred_element_type=jnp.float32) + jnp.dot(
        dv, wevt_ref[...], preferred_element_type=jnp.float32)


@jax.custom_vjp
def _attn(xs, qd, ef, wk, wv, wek, wev):
    e = xs.shape[0]
    te, nb = _edge_grid(e)
    return pl.pallas_call(
        _attn_fwd_body,
        grid=(nb,),
        in_specs=[
            _espec(te, _C), _espec(te, _DATTN), _espec(te, _C),
            _wspec((_C, _DATTN)), _wspec((_C, _DATTN)),
            _wspec((_C, _DATTN)), _wspec((_C, _DATTN)),
        ],
        out_specs=[_espec(te, _HEADS), _espec(te, _DATTN)],
        out_shape=[
            jax.ShapeDtypeStruct((e, _HEADS), jnp.float32),
            jax.ShapeDtypeStruct((e, _DATTN), jnp.float32),
        ],
    )(xs, qd, ef, wk, wv, wek, wev)


def _attn_vfwd(xs, qd, ef, wk, wv, wek, wev):
    return _attn(xs, qd, ef, wk, wv, wek, wev), (xs, qd, ef, wk, wv, wek, wev)


def _attn_vbwd(res, ct):
    xs, qd, ef, wk, wv, wek, wev = res
    dlogits, dv = ct
    e = xs.shape[0]
    te, nb = _edge_grid(e)
    dxs, dqd, def_ = pl.pallas_call(
        _attn_bwd_body,
        grid=(nb,),
        in_specs=[
            _espec(te, _C), _espec(te, _DATTN), _espec(te, _C),
            _wspec((_C, _DATTN)), _wspec((_C, _DATTN)),
            _wspec((_DATTN, _C)), _wspec((_DATTN, _C)),
            _wspec((_DATTN, _C)), _wspec((_DATTN, _C)),
            _espec(te, _HEADS), _espec(te, _DATTN),
        ],
        out_specs=[_espec(te, _C), _espec(te, _DATTN), _espec(te, _C)],
        out_shape=[
            jax.ShapeDtypeStruct((e, _C), jnp.float32),
            jax.ShapeDtypeStruct((e, _DATTN), jnp.float32),
            jax.ShapeDtypeStruct((e, _C), jnp.float32),
        ],
    )(xs, qd, ef, wk, wek, wv.T, wev.T, wk.T, wek.T, dlogits, dv)
    return (dxs, dqd, def_, jnp.zeros_like(wk), jnp.zeros_like(wv),
            jnp.zeros_like(wek), jnp.zeros_like(wev))


_attn.defvjp(_attn_vfwd, _attn_vbwd)


def _soft_fwd_body(scale_ref, logits_ref, v_ref, numw_ref):
    e = jnp.exp(logits_ref[...])
    num = scale_ref[...] * e
    numv = jnp.dot(num, _m8x16(), preferred_element_type=jnp.float32) * v_ref[...]
    numw_ref[...] = jnp.concatenate([num, numv], axis=1)


def _soft_bwd_body(scale_ref, logits_ref, v_ref, ct_ref,
                   dscale_ref, dlogits_ref, dv_ref):
    e = jnp.exp(logits_ref[...])
    num = scale_ref[...] * e
    ct = ct_ref[...]
    dnum = ct[:, :_HEADS]
    dnumv = ct[:, _HEADS:]
    dv_ref[...] = jnp.dot(num, _m8x16(), preferred_element_type=jnp.float32) * dnumv
    dnum_tot = dnum + jnp.dot(
        dnumv * v_ref[...], _m16x8(), preferred_element_type=jnp.float32)
    dlogits_ref[...] = dnum_tot * num
    dscale_ref[...] = jnp.sum(dnum_tot * e, axis=1, keepdims=True)


@jax.custom_vjp
def _soft(scale, logits, v):
    e = logits.shape[0]
    te, nb = _edge_grid(e)
    return pl.pallas_call(
        _soft_fwd_body,
        grid=(nb,),
        in_specs=[_espec(te, 1), _espec(te, _HEADS), _espec(te, _DATTN)],
        out_specs=_espec(te, _HEADS + _DATTN),
        out_shape=jax.ShapeDtypeStruct((e, _HEADS + _DATTN), jnp.float32),
    )(scale, logits, v)


def _soft_vfwd(scale, logits, v):
    return _soft(scale, logits, v), (scale, logits, v)


def _soft_vbwd(res, ct):
    scale, logits, v = res
    e = logits.shape[0]
    te, nb = _edge_grid(e)
    dscale, dlogits, dv = pl.pallas_call(
        _soft_bwd_body,
        grid=(nb,),
        in_specs=[_espec(te, 1), _espec(te, _HEADS),
                  _espec(te, _DATTN), _espec(te, _HEADS + _DATTN)],
        out_specs=[_espec(te, 1), _espec(te, _HEADS), _espec(te, _DATTN)],
        out_shape=[
            jax.ShapeDtypeStruct((e, 1), jnp.float32),
            jax.ShapeDtypeStruct((e, _HEADS), jnp.float32),
            jax.ShapeDtypeStruct((e, _DATTN), jnp.float32),
        ],
    )(scale, logits, v, ct)
    return (dscale, dlogits, dv)


_soft.defvjp(_soft_vfwd, _soft_vbwd)


def _final_fwd_body(xs_ref, ef_ref, scale_ref, won_ref, woe_ref, w1_ref,
                    b1_ref, w2_ref, b2_ref, out_ref):
    fe = jnp.dot(xs_ref[...], won_ref[...], preferred_element_type=jnp.float32) * jnp.dot(
        ef_ref[...], woe_ref[...], preferred_element_type=jnp.float32)
    z1 = jnp.dot(fe, w1_ref[...], preferred_element_type=jnp.float32) + b1_ref[...]
    h = z1 / (1.0 + jnp.exp(-z1))
    z2 = jnp.dot(h, w2_ref[...], preferred_element_type=jnp.float32) + b2_ref[...]
    s = jnp.sum(z2 * scale_ref[...], keepdims=True)

    @pl.when(pl.program_id(0) == 0)
    def _():
        out_ref[...] = jnp.zeros((1, 1), jnp.float32)

    out_ref[...] += s


def _final_bwd_body(xs_ref, ef_ref, scale_ref, won_ref, woe_ref, w1_ref,
                    b1_ref, w2_ref, b2_ref, w1t_ref, wont_ref, woet_ref,
                    g_ref, dxs_ref, def_ref, dscale_ref):
    a = jnp.dot(xs_ref[...], won_ref[...], preferred_element_type=jnp.float32)
    b = jnp.dot(ef_ref[...], woe_ref[...], preferred_element_type=jnp.float32)
    fe = a * b
    z1 = jnp.dot(fe, w1_ref[...], preferred_element_type=jnp.float32) + b1_ref[...]
    sig = 1.0 / (1.0 + jnp.exp(-z1))
    h = z1 * sig
    z2 = jnp.dot(h, w2_ref[...], preferred_element_type=jnp.float32) + b2_ref[...]
    g = g_ref[...]
    de = g * scale_ref[...]
    dscale_ref[...] = g * z2
    dh = de * jnp.transpose(w2_ref[...])
    dz1 = dh * (sig * (1.0 + z1 * (1.0 - sig)))
    dfe = jnp.dot(dz1, w1t_ref[...], preferred_element_type=jnp.float32)
    dxs_ref[...] = jnp.dot(dfe * b, wont_ref[...], preferred_element_type=jnp.float32)
    def_ref[...] = jnp.dot(dfe * a, woet_ref[...], preferred_element_type=jnp.float32)


@jax.custom_vjp
def _final(xs, ef, scale, won, woe, w1, b1, w2, b2):
    e = xs.shape[0]
    te, nb = _edge_grid(e)
    out = pl.pallas_call(
        _final_fwd_body,
        grid=(nb,),
        in_specs=[
            _espec(te, _C), _espec(te, _C), _espec(te, 1),
            _wspec((_C, _OUTC)), _wspec((_C, _OUTC)),
            _wspec((_OUTC, _OUTC)), _wspec((1, _OUTC)),
            _wspec((_OUTC, 1)), _wspec((1, 1)),
        ],
        out_specs=pl.BlockSpec((1, 1), lambda i: (0, 0)),
        out_shape=jax.ShapeDtypeStruct((1, 1), jnp.float32),
    )(xs, ef, scale, won, woe, w1, b1[None, :], w2, b2[None, :])
    return out[0, 0]


def _final_vfwd(xs, ef, scale, won, woe, w1, b1, w2, b2):
    return _final(xs, ef, scale, won, woe, w1, b1, w2, b2), (
        xs, ef, scale, won, woe, w1, b1, w2, b2)


def _final_vbwd(res, g):
    xs, ef, scale, won, woe, w1, b1, w2, b2 = res
    e = xs.shape[0]
    te, nb = _edge_grid(e)
    dxs, def_, dscale = pl.pallas_call(
        _final_bwd_body,
        grid=(nb,),
        in_specs=[
            _espec(te, _C), _espec(te, _C), _espec(te, 1),
            _wspec((_C, _OUTC)), _wspec((_C, _OUTC)),
            _wspec((_OUTC, _OUTC)), _wspec((1, _OUTC)),
            _wspec((_OUTC, 1)), _wspec((1, 1)),
            _wspec((_OUTC, _OUTC)), _wspec((_OUTC, _C)), _wspec((_OUTC, _C)),
            _wspec((1, 1)),
        ],
        out_specs=[_espec(te, _C), _espec(te, _C), _espec(te, 1)],
        out_shape=[
            jax.ShapeDtypeStruct((e, _C), jnp.float32),
            jax.ShapeDtypeStruct((e, _C), jnp.float32),
            jax.ShapeDtypeStruct((e, 1), jnp.float32),
        ],
    )(xs, ef, scale, won, woe, w1, b1[None, :], w2, b2[None, :],
      w1.T, won.T, woe.T, jnp.reshape(g, (1, 1)).astype(jnp.float32))
    return (dxs, def_, dscale, jnp.zeros_like(won), jnp.zeros_like(woe),
            jnp.zeros_like(w1), jnp.zeros_like(b1), jnp.zeros_like(w2),
            jnp.zeros_like(b2))


_final.defvjp(_final_vfwd, _final_vbwd)


def _energy_impl(pos, species, src, dst, params):
    n = pos.shape[0]
    gather = _gather_fn(n)
    segsum = _segsum_fn(n)
    pos4 = jnp.concatenate([pos, jnp.zeros((n, 5), jnp.float32)], axis=1)
    rel = gather(pos4, dst) - gather(pos4, src)
    scale, ef = _geom(rel)
    x = jnp.take(params['embedding'], species - 1, axis=0)
    for lp in params['layers']:
        q = x @ lp['Wq']
        xs = gather(x, src)
        qd = gather(q, dst)
        logits, v = _attn(xs, qd, ef, lp['Wk'], lp['Wv'], lp['Wek'], lp['Wev'])
        numw = _soft(scale, logits, v)
        segw = segsum(numw, dst)
        den = segw[:, :_HEADS]
        sv = segw[:, _HEADS:]
        agg = (sv.reshape(n, _HEADS, _HD)
               / (den[..., None] + 1e-9)).reshape(n, _DATTN)
        x = agg @ lp['Wo'] + x @ lp['Wskip']
        mu = jnp.mean(x, axis=-1, keepdims=True)
        var = jnp.var(x, axis=-1, keepdims=True)
        x = (x - mu) / jnp.sqrt(var + 1e-5) * lp['gamma'] + lp['beta']
    xs = gather(x, src)
    return _final(xs, ef, scale, params['Won'], params['Woe'],
                  params['mlp_w1'], params['mlp_b1'],
                  params['mlp_w2'], params['mlp_b2'])


def kernel(pos, species, edge_index, params):
    src = edge_index[0]
    dst = edge_index[1]

    def efn(p):
        return _energy_impl(p, species, src, dst, params)

    energy, dpos = jax.value_and_grad(efn)(pos)
    return energy, -dpos
